# Initial kernel scaffold; baseline (speedup 1.0000x reference)
import jax
import jax.numpy as jnp
from jax.experimental import pallas as pl


def kernel(h, span_idx, W1, b1, W2, b2):
    raise NotImplementedError("write your pallas kernel here")


# speedup vs baseline: 399.6032x; 399.6032x over previous
import jax
import jax.numpy as jnp
from jax.experimental import pallas as pl
from jax.experimental.pallas import tpu as pltpu

HIDDEN = 256
B, L, NUM_SPANS = 4, 512, 256


def _span_mlp_kernel(idx_ref, h_ref, w1_ref, b1_ref, w2_ref, b2_ref, out_ref):
    starts = idx_ref[0, :, 0]
    ends = idx_ref[0, :, 1]
    pos = jax.lax.broadcasted_iota(jnp.int32, (NUM_SPANS, L), 1)
    mask = ((pos >= starts[:, None]) & (pos <= ends[:, None])).astype(jnp.float32)
    hb = h_ref[0]
    sums = jax.lax.dot_general(
        mask, hb, (((1,), (0,)), ((), ())),
        precision=jax.lax.Precision.HIGHEST,
        preferred_element_type=jnp.float32,
    )
    lengths = (ends - starts + 1).astype(jnp.float32)
    reps = sums * (1.0 / lengths)[:, None]
    x = jax.lax.dot_general(
        reps, w1_ref[...], (((1,), (0,)), ((), ())),
        precision=jax.lax.Precision.HIGHEST,
        preferred_element_type=jnp.float32,
    )
    x = jnp.maximum(x + b1_ref[...], 0.0)
    out = jax.lax.dot_general(
        x, w2_ref[...], (((1,), (0,)), ((), ())),
        precision=jax.lax.Precision.HIGHEST,
        preferred_element_type=jnp.float32,
    )
    out_ref[0] = out + b2_ref[...]


def kernel(h, span_idx, W1, b1, W2, b2):
    span_idx = span_idx.astype(jnp.int32)
    b1 = b1.reshape(1, 4 * HIDDEN)
    b2 = b2.reshape(1, HIDDEN)
    out = pl.pallas_call(
        _span_mlp_kernel,
        grid=(B,),
        in_specs=[
            pl.BlockSpec((1, NUM_SPANS, 2), lambda b: (b, 0, 0)),
            pl.BlockSpec((1, L, HIDDEN), lambda b: (b, 0, 0)),
            pl.BlockSpec((HIDDEN, 4 * HIDDEN), lambda b: (0, 0)),
            pl.BlockSpec((1, 4 * HIDDEN), lambda b: (0, 0)),
            pl.BlockSpec((4 * HIDDEN, HIDDEN), lambda b: (0, 0)),
            pl.BlockSpec((1, HIDDEN), lambda b: (0, 0)),
        ],
        out_specs=pl.BlockSpec((1, NUM_SPANS, HIDDEN), lambda b: (b, 0, 0)),
        out_shape=jax.ShapeDtypeStruct((B, NUM_SPANS, HIDDEN), jnp.float32),
    )(span_idx, h, W1, b1, W2, b2)
    return out


# speedup vs baseline: 665.8017x; 1.6662x over previous
import jax
import jax.numpy as jnp
from jax.experimental import pallas as pl
from jax.experimental.pallas import tpu as pltpu

HIDDEN = 256
B, L, NUM_SPANS = 4, 512, 256


def _span_mlp_kernel(idx_ref, h_ref, w1_ref, b1_ref, w2_ref, b2_ref, out_ref):
    starts = idx_ref[0, :, 0]
    ends = idx_ref[0, :, 1]
    pos = jax.lax.broadcasted_iota(jnp.int32, (NUM_SPANS, L), 1)
    mask = ((pos >= starts[:, None]) & (pos <= ends[:, None])).astype(jnp.float32)
    hb = h_ref[0]
    sums = jax.lax.dot_general(
        mask, hb, (((1,), (0,)), ((), ())),
        precision=jax.lax.Precision.HIGHEST,
        preferred_element_type=jnp.float32,
    )
    lengths = (ends - starts + 1).astype(jnp.float32)
    reps = sums * (1.0 / lengths)[:, None]
    x = jax.lax.dot_general(
        reps, w1_ref[...], (((1,), (0,)), ((), ())),
        precision=jax.lax.Precision.DEFAULT,
        preferred_element_type=jnp.float32,
    )
    x = jnp.maximum(x + b1_ref[...], 0.0)
    out = jax.lax.dot_general(
        x, w2_ref[...], (((1,), (0,)), ((), ())),
        precision=jax.lax.Precision.DEFAULT,
        preferred_element_type=jnp.float32,
    )
    out_ref[0] = out + b2_ref[...]


def kernel(h, span_idx, W1, b1, W2, b2):
    span_idx = span_idx.astype(jnp.int32)
    b1 = b1.reshape(1, 4 * HIDDEN)
    b2 = b2.reshape(1, HIDDEN)
    out = pl.pallas_call(
        _span_mlp_kernel,
        grid=(B,),
        in_specs=[
            pl.BlockSpec((1, NUM_SPANS, 2), lambda b: (b, 0, 0)),
            pl.BlockSpec((1, L, HIDDEN), lambda b: (b, 0, 0)),
            pl.BlockSpec((HIDDEN, 4 * HIDDEN), lambda b: (0, 0)),
            pl.BlockSpec((1, 4 * HIDDEN), lambda b: (0, 0)),
            pl.BlockSpec((4 * HIDDEN, HIDDEN), lambda b: (0, 0)),
            pl.BlockSpec((1, HIDDEN), lambda b: (0, 0)),
        ],
        out_specs=pl.BlockSpec((1, NUM_SPANS, HIDDEN), lambda b: (b, 0, 0)),
        out_shape=jax.ShapeDtypeStruct((B, NUM_SPANS, HIDDEN), jnp.float32),
    )(span_idx, h, W1, b1, W2, b2)
    return out
